# Initial kernel scaffold; baseline (speedup 1.0000x reference)
import functools

import jax
import jax.numpy as jnp
from jax import lax
from jax.experimental import pallas as pl
from jax.experimental.pallas import tpu as pltpu
from jax.experimental.pallas import tpu_sc as plsc

NC = 2
NS = 16
LN = 16
NW = NC * NS


def _fill_f32(ref, rows, cols, val):
    cpr = cols // LN

    def body(i, carry):
        r = i // cpr
        c = (i % cpr) * LN
        ref[r, pl.ds(c, LN)] = jnp.full((LN,), val, dtype=jnp.float32)
        return carry

    lax.fori_loop(0, rows * cpr, body, 0)


def _pick_chunk(epw):
    for c in range(128, 7, -8):
        if epw % c == 0:
            return c
    return None


@functools.lru_cache(maxsize=None)
def _make_seg_sum(n, e, feat, with_deg):
    epw = e // NW
    C = _pick_chunk(epw)
    nchunks = epw // C
    rpt = n // NS
    zrows = 125 if rpt % 125 == 0 else rpt
    nz = rpt // zrows

    mesh = plsc.VectorSubcoreMesh(
        core_axis_name="c", subcore_axis_name="s",
        num_cores=NC, num_subcores=NS)

    out_type = [jax.ShapeDtypeStruct((NC, n, feat), jnp.float32)]
    scratch = [
        pltpu.VMEM((2, C), jnp.int32),
        pltpu.VMEM((2, C), jnp.int32),
        pltpu.VMEM((C, feat), jnp.float32),
        pltpu.VMEM((zrows, feat), jnp.float32),
        pltpu.VMEM_SHARED((n, feat), jnp.float32),
        pltpu.SemaphoreType.DMA,
    ]
    if with_deg:
        out_type.append(jax.ShapeDtypeStruct((NC, n, LN), jnp.float32))
        scratch += [
            pltpu.VMEM((C, LN), jnp.float32),
            pltpu.VMEM((rpt, LN), jnp.float32),
            pltpu.VMEM_SHARED((n, LN), jnp.float32),
        ]

    def body(x_hbm, src_hbm, dst_hbm, *refs):
        if with_deg:
            (out_hbm, deg_hbm, idx_s, idx_d, rows, zbuf, acc, sem,
             ones, zdeg, dacc) = refs
        else:
            out_hbm, idx_s, idx_d, rows, zbuf, acc, sem = refs
        cid = lax.axis_index("c")
        sid = lax.axis_index("s")
        wid = cid * NS + sid
        base = wid * epw

        _fill_f32(zbuf, zrows, feat, 0.0)
        for k in range(nz):
            pltpu.sync_copy(zbuf, acc.at[pl.ds(sid * rpt + k * zrows, zrows)])
        if with_deg:
            _fill_f32(ones, C, LN, 1.0)
            _fill_f32(zdeg, rpt, LN, 0.0)
            pltpu.sync_copy(zdeg, dacc.at[pl.ds(sid * rpt, rpt)])
        plsc.subcore_barrier()

        def chunk(g, carry):
            off = pl.multiple_of(base + g * C, 8)
            pltpu.sync_copy(src_hbm.at[pl.ds(off, C)], idx_s.at[0])
            pltpu.sync_copy(dst_hbm.at[pl.ds(off, C)], idx_d.at[0])
            pltpu.async_copy(x_hbm.at[idx_s.at[0]], rows, sem).wait()
            pltpu.sync_copy(rows, acc.at[idx_d.at[0]], add=True)
            if with_deg:
                pltpu.sync_copy(ones, dacc.at[idx_d.at[0]], add=True)
            return carry

        lax.fori_loop(0, nchunks, chunk, 0)
        plsc.subcore_barrier()

        pltpu.sync_copy(acc.at[pl.ds(sid * rpt, rpt)],
                        out_hbm.at[cid, pl.ds(sid * rpt, rpt)])
        if with_deg:
            pltpu.sync_copy(dacc.at[pl.ds(sid * rpt, rpt)],
                            deg_hbm.at[cid, pl.ds(sid * rpt, rpt)])

    return pl.kernel(body, out_type=out_type, mesh=mesh,
                     scratch_types=scratch)


@functools.lru_cache(maxsize=None)
def _make_layer(n, feat):
    def body(x_ref, s_ref, d_ref, wl_ref, wr_ref, b_ref, o_ref):
        s = s_ref[0] + s_ref[1]
        d = d_ref[0, :, 0:1] + d_ref[1, :, 0:1]
        mean = s / jnp.maximum(d, 1.0)
        acc = jax.lax.dot_general(
            mean, wl_ref[...], (((1,), (0,)), ((), ())),
            precision=lax.Precision.HIGHEST,
            preferred_element_type=jnp.float32)
        acc = acc + jax.lax.dot_general(
            x_ref[...], wr_ref[...], (((1,), (0,)), ((), ())),
            precision=lax.Precision.HIGHEST,
            preferred_element_type=jnp.float32)
        o_ref[...] = jnp.tanh(acc + b_ref[...])

    return pl.pallas_call(
        body, out_shape=jax.ShapeDtypeStruct((n, feat), jnp.float32))


def kernel(x, edge_index, W1l, W1r, b1, W2l, W2r, b2):
    n, feat = x.shape
    e = edge_index.shape[1]
    ei = edge_index.astype(jnp.int32)
    src = ei[0]
    dst = ei[1]

    seg_deg = _make_seg_sum(n, e, feat, True)
    seg = _make_seg_sum(n, e, feat, False)
    layer = _make_layer(n, feat)

    sums1, degp = seg_deg(x, src, dst)
    h = layer(x, sums1, degp, W1l, W1r, b1.reshape(1, feat))
    (sums2,) = seg(h, src, dst)
    out = layer(h, sums2, degp, W2l, W2r, b2.reshape(1, feat))
    return out


# speedup vs baseline: 4.5542x; 4.5542x over previous
import functools

import jax
import jax.numpy as jnp
from jax import lax
from jax.experimental import pallas as pl
from jax.experimental.pallas import tpu as pltpu
from jax.experimental.pallas import tpu_sc as plsc

NC = 2
NS = 16
LN = 16
NW = NC * NS
BLK = 128


def _fill_f32(ref, rows, cols, val):
    cpr = cols // LN

    def body(i, carry):
        r = i // cpr
        c = (i % cpr) * LN
        ref[r, pl.ds(c, LN)] = jnp.full((LN,), val, dtype=jnp.float32)
        return carry

    lax.fori_loop(0, rows * cpr, body, 0)


def _pick_chunk(epw):
    for c in range(128, 7, -8):
        if epw % c == 0:
            return c
    raise ValueError(f"no chunk size for {epw} edges per worker")


@functools.lru_cache(maxsize=None)
def _make_seg_sum(n, e, feat, with_deg):
    epw = e // NW
    C = _pick_chunk(epw)
    nchunks = epw // C
    dp = (-(-n // BLK) + LN - 1) // LN * LN
    rpt = (n // (NS * 8)) * 8
    tail = n - NS * rpt
    zrows = rpt
    for cand in (104, 125, 78, 100):
        if rpt % cand == 0 and cand >= tail:
            zrows = cand
            break
    nz = rpt // zrows

    mesh = plsc.VectorSubcoreMesh(
        core_axis_name="c", subcore_axis_name="s",
        num_cores=NC, num_subcores=NS)

    sums_type = jax.ShapeDtypeStruct((NC, n, feat), jnp.float32)
    out_type = [sums_type]
    scratch = [
        pltpu.VMEM((2, C), jnp.int32),
        pltpu.VMEM((2, C), jnp.int32),
        pltpu.VMEM((C, feat), jnp.float32),
        pltpu.VMEM((zrows, feat), jnp.float32),
        pltpu.VMEM_SHARED((n, feat), jnp.float32),
        pltpu.SemaphoreType.DMA,
    ]
    if with_deg:
        out_type.append(
            jax.ShapeDtypeStruct((NC, NS, dp * BLK), jnp.float32))
        scratch += [
            pltpu.VMEM((dp * BLK,), jnp.float32),
        ]

    def body(x_hbm, src_hbm, dst_hbm, *refs):
        if with_deg:
            (out_hbm, deg_hbm, idx_s, idx_d, rows, zbuf, acc, sem,
             dvm) = refs
        else:
            out_hbm, idx_s, idx_d, rows, zbuf, acc, sem = refs
        cid = lax.axis_index("c")
        sid = lax.axis_index("s")
        wid = cid * NS + sid
        base = wid * epw

        _fill_f32(zbuf, zrows, feat, 0.0)
        for k in range(nz):
            pltpu.sync_copy(zbuf, acc.at[pl.ds(sid * rpt + k * zrows, zrows)])
        if tail:
            @pl.when(sid == NS - 1)
            def _zero_tail():
                pltpu.sync_copy(zbuf.at[pl.ds(0, tail)],
                                acc.at[pl.ds(NS * rpt, tail)])
        if with_deg:
            def zb(i, carry):
                dvm[pl.ds(i * LN, LN)] = jnp.zeros((LN,), jnp.float32)
                return carry
            lax.fori_loop(0, dp * BLK // LN, zb, 0)
        plsc.subcore_barrier()

        one16 = jnp.full((LN,), 1.0, dtype=jnp.float32)

        def chunk(g, carry):
            off = pl.multiple_of(base + g * C, 8)
            pltpu.sync_copy(src_hbm.at[pl.ds(off, C)], idx_s.at[0])
            pltpu.sync_copy(dst_hbm.at[pl.ds(off, C)], idx_d.at[0])
            pltpu.async_copy(x_hbm.at[idx_s.at[0]], rows, sem).wait()
            pltpu.sync_copy(rows, acc.at[idx_d.at[0]], add=True)
            if with_deg:
                for k in range(C // LN):
                    di = idx_d[0, pl.ds(k * LN, LN)]
                    plsc.addupdate_scatter(dvm, [di], one16)
            return carry

        lax.fori_loop(0, nchunks, chunk, 0)
        if with_deg:
            pltpu.sync_copy(dvm, deg_hbm.at[cid, sid])
        plsc.subcore_barrier()

        pltpu.sync_copy(acc.at[pl.ds(sid * rpt, rpt)],
                        out_hbm.at[cid, pl.ds(sid * rpt, rpt)])
        if tail:
            @pl.when(sid == NS - 1)
            def _write_tail():
                pltpu.sync_copy(acc.at[pl.ds(NS * rpt, tail)],
                                out_hbm.at[cid, pl.ds(NS * rpt, tail)])

    return pl.kernel(body,
                     out_type=tuple(out_type) if with_deg else sums_type,
                     mesh=mesh, scratch_types=tuple(scratch),
                     compiler_params=pltpu.CompilerParams(
                         needs_layout_passes=False))


@functools.lru_cache(maxsize=None)
def _make_layer(n, feat):
    nb = -(-n // BLK)

    def body(x_ref, s_ref, d_ref, wl_ref, wr_ref, b_ref, o_ref):
        s = s_ref[0] + s_ref[1]
        drow = jnp.sum(d_ref[...], axis=0)[0]
        r = lax.broadcasted_iota(jnp.int32, (BLK, BLK), 0)
        c = lax.broadcasted_iota(jnp.int32, (BLK, BLK), 1)
        eye = (r == c).astype(jnp.float32)
        dcol = jax.lax.dot_general(
            eye, drow, (((1,), (1,)), ((), ())),
            preferred_element_type=jnp.float32)
        mean = s / jnp.maximum(dcol, 1.0)
        acc = jax.lax.dot_general(
            mean, wl_ref[...], (((1,), (0,)), ((), ())),
            precision=lax.Precision.HIGHEST,
            preferred_element_type=jnp.float32)
        acc = acc + jax.lax.dot_general(
            x_ref[...], wr_ref[...], (((1,), (0,)), ((), ())),
            precision=lax.Precision.HIGHEST,
            preferred_element_type=jnp.float32)
        o_ref[...] = jnp.tanh(acc + b_ref[...])

    return pl.pallas_call(
        body,
        grid=(nb,),
        in_specs=[
            pl.BlockSpec((BLK, feat), lambda b: (b, 0)),
            pl.BlockSpec((NC, BLK, feat), lambda b: (0, b, 0)),
            pl.BlockSpec((NW, 1, 1, BLK), lambda b: (0, b, 0, 0)),
            pl.BlockSpec((feat, feat), lambda b: (0, 0)),
            pl.BlockSpec((feat, feat), lambda b: (0, 0)),
            pl.BlockSpec((1, feat), lambda b: (0, 0)),
        ],
        out_specs=pl.BlockSpec((BLK, feat), lambda b: (b, 0)),
        out_shape=jax.ShapeDtypeStruct((n, feat), jnp.float32))


def kernel(x, edge_index, W1l, W1r, b1, W2l, W2r, b2):
    n, feat = x.shape
    e = edge_index.shape[1]
    ei = edge_index.astype(jnp.int32)
    src = ei[0]
    dst = ei[1]
    seg_deg = _make_seg_sum(n, e, feat, True)
    seg = _make_seg_sum(n, e, feat, False)
    layer = _make_layer(n, feat)

    sums1, degp = seg_deg(x, src, dst)
    degp = degp.reshape(NW, degp.shape[2] // BLK, 1, BLK)
    h = layer(x, sums1, degp, W1l, W1r, b1.reshape(1, feat))
    sums2 = seg(h, src, dst)
    out = layer(h, sums2, degp, W2l, W2r, b2.reshape(1, feat))
    return out
